# baseline (speedup 1.0000x reference)
import functools

import jax
import jax.numpy as jnp
from jax import lax
from jax.experimental import pallas as pl
from jax.experimental.pallas import tpu as pltpu
from jax.experimental.pallas import tpu_sc as plsc

_T = 1000
_TPAD = 1008
_B = 16384
_L = 16
_NC, _NS = 2, 16
_NW = _NC * _NS
_CHUNK = _B // _NW


def _lane_perm(x, perm):
    dnums = lax.GatherDimensionNumbers(
        offset_dims=(), collapsed_slice_dims=(0,), start_index_map=(0,))
    return lax.gather(x, perm[:, None], dnums, slice_sizes=(1,),
                      mode=lax.GatherScatterMode.PROMISE_IN_BOUNDS)


def _sc_body(steps_hbm, betas_hbm, out_hbm, betas_v, ab_v, idx_v, ob_v, oa_v, op_v):
    wid = lax.axis_index("s") * _NC + lax.axis_index("c")
    base = wid * _CHUNK

    pltpu.sync_copy(betas_hbm, betas_v.at[pl.ds(0, _T)])
    pltpu.sync_copy(steps_hbm.at[pl.ds(base, _CHUNK)], idx_v)

    lane = lax.iota(jnp.int32, _L)
    shifts = [(jnp.maximum(lane - s, 0), lane >= s) for s in (1, 2, 4, 8)]
    last = jnp.full((_L,), _L - 1, jnp.int32)

    def prod_body(j, carry):
        v = 1.0 - betas_v[pl.ds(j * _L, _L)]
        for perm, mask in shifts:
            v = v * jnp.where(mask, _lane_perm(v, perm), 1.0)
        v = v * carry
        ab_v[pl.ds(j * _L, _L)] = v
        return _lane_perm(v, last)

    lax.fori_loop(0, _TPAD // _L, prod_body, jnp.ones((_L,), jnp.float32))

    def gather_body(i, _):
        iv = idx_v[pl.ds(i * _L, _L)]
        b = plsc.load_gather(betas_v, [iv])
        p = plsc.load_gather(ab_v, [iv])
        ob_v[pl.ds(i * _L, _L)] = b
        oa_v[pl.ds(i * _L, _L)] = 1.0 - b
        op_v[pl.ds(i * _L, _L)] = p
        return 0

    lax.fori_loop(0, _CHUNK // _L, gather_body, 0)

    pltpu.sync_copy(ob_v, out_hbm.at[0, pl.ds(base, _CHUNK)])
    pltpu.sync_copy(oa_v, out_hbm.at[1, pl.ds(base, _CHUNK)])
    pltpu.sync_copy(op_v, out_hbm.at[2, pl.ds(base, _CHUNK)])


_sc_lookup = functools.partial(
    pl.kernel,
    mesh=plsc.VectorSubcoreMesh(core_axis_name="c", subcore_axis_name="s"),
    compiler_params=pltpu.CompilerParams(
        needs_layout_passes=False, use_tc_tiling_on_sc=False),
    out_type=jax.ShapeDtypeStruct((3, _B), jnp.float32),
    scratch_types=[
        pltpu.VMEM((_TPAD,), jnp.float32),
        pltpu.VMEM((_TPAD,), jnp.float32),
        pltpu.VMEM((_CHUNK,), jnp.int32),
        pltpu.VMEM((_CHUNK,), jnp.float32),
        pltpu.VMEM((_CHUNK,), jnp.float32),
        pltpu.VMEM((_CHUNK,), jnp.float32),
    ],
)(_sc_body)


def kernel(num_steps, betas):
    return _sc_lookup(num_steps.astype(jnp.int32), betas)


# speedup vs baseline: 1.0275x; 1.0275x over previous
import functools

import jax
import jax.numpy as jnp
from jax import lax
from jax.experimental import pallas as pl
from jax.experimental.pallas import tpu as pltpu
from jax.experimental.pallas import tpu_sc as plsc

_T = 1000
_TPAD = 1008
_B = 16384
_L = 16
_NC, _NS = 2, 16
_NW = _NC * _NS
_CHUNK = _B // _NW


def _lane_perm(x, perm):
    dnums = lax.GatherDimensionNumbers(
        offset_dims=(), collapsed_slice_dims=(0,), start_index_map=(0,))
    return lax.gather(x, perm[:, None], dnums, slice_sizes=(1,),
                      mode=lax.GatherScatterMode.PROMISE_IN_BOUNDS)


def _sc_body(steps_hbm, betas_hbm, out_hbm, betas_v, ab_v, idx_v, ob_v, oa_v, op_v,
             sem0, sem1, sem2):
    wid = lax.axis_index("s") * _NC + lax.axis_index("c")
    base = wid * _CHUNK

    cp_betas = pltpu.async_copy(betas_hbm, betas_v.at[pl.ds(0, _T)], sem0)
    cp_idx = pltpu.async_copy(steps_hbm.at[pl.ds(base, _CHUNK)], idx_v, sem1)
    cp_betas.wait()

    lane = lax.iota(jnp.int32, _L)
    shifts = [(jnp.maximum(lane - s, 0), lane >= s) for s in (1, 2, 4, 8)]
    last = jnp.full((_L,), _L - 1, jnp.int32)

    def prod_body(j, carry):
        v = 1.0 - betas_v[pl.ds(j * _L, _L)]
        for perm, mask in shifts:
            v = v * jnp.where(mask, _lane_perm(v, perm), 1.0)
        v = v * carry
        ab_v[pl.ds(j * _L, _L)] = v
        return _lane_perm(v, last)

    lax.fori_loop(0, _TPAD // _L, prod_body, jnp.ones((_L,), jnp.float32))
    cp_idx.wait()

    def gather_body(i, _):
        iv = idx_v[pl.ds(i * _L, _L)]
        b = plsc.load_gather(betas_v, [iv])
        p = plsc.load_gather(ab_v, [iv])
        ob_v[pl.ds(i * _L, _L)] = b
        oa_v[pl.ds(i * _L, _L)] = 1.0 - b
        op_v[pl.ds(i * _L, _L)] = p
        return 0

    lax.fori_loop(0, _CHUNK // _L, gather_body, 0)

    cp0 = pltpu.async_copy(ob_v, out_hbm.at[0, pl.ds(base, _CHUNK)], sem0)
    cp1 = pltpu.async_copy(oa_v, out_hbm.at[1, pl.ds(base, _CHUNK)], sem1)
    cp2 = pltpu.async_copy(op_v, out_hbm.at[2, pl.ds(base, _CHUNK)], sem2)
    cp0.wait()
    cp1.wait()
    cp2.wait()


_sc_lookup = functools.partial(
    pl.kernel,
    mesh=plsc.VectorSubcoreMesh(core_axis_name="c", subcore_axis_name="s"),
    compiler_params=pltpu.CompilerParams(
        needs_layout_passes=False, use_tc_tiling_on_sc=False),
    out_type=jax.ShapeDtypeStruct((3, _B), jnp.float32),
    scratch_types=[
        pltpu.VMEM((_TPAD,), jnp.float32),
        pltpu.VMEM((_TPAD,), jnp.float32),
        pltpu.VMEM((_CHUNK,), jnp.int32),
        pltpu.VMEM((_CHUNK,), jnp.float32),
        pltpu.VMEM((_CHUNK,), jnp.float32),
        pltpu.VMEM((_CHUNK,), jnp.float32),
        pltpu.SemaphoreType.DMA,
        pltpu.SemaphoreType.DMA,
        pltpu.SemaphoreType.DMA,
    ],
)(_sc_body)


def kernel(num_steps, betas):
    return _sc_lookup(num_steps.astype(jnp.int32), betas)


# speedup vs baseline: 1.1140x; 1.0841x over previous
import functools

import jax
import jax.numpy as jnp
from jax import lax
from jax.experimental import pallas as pl
from jax.experimental.pallas import tpu as pltpu
from jax.experimental.pallas import tpu_sc as plsc

_T = 1000
_TPAD = 1008
_B = 16384
_L = 16
_NC, _NS = 2, 16
_NW = _NC * _NS
_CHUNK = _B // _NW


def _lane_perm(x, perm):
    dnums = lax.GatherDimensionNumbers(
        offset_dims=(), collapsed_slice_dims=(0,), start_index_map=(0,))
    return lax.gather(x, perm[:, None], dnums, slice_sizes=(1,),
                      mode=lax.GatherScatterMode.PROMISE_IN_BOUNDS)


def _sc_body(steps_hbm, betas_hbm, out_hbm, betas_v, ab_v, idx_v, ob_v, oa_v, op_v,
             sem0, sem1, sem2):
    wid = lax.axis_index("s") * _NC + lax.axis_index("c")
    base = wid * _CHUNK

    cp_betas = pltpu.async_copy(betas_hbm, betas_v.at[pl.ds(0, _T)], sem0)
    cp_idx = pltpu.async_copy(steps_hbm.at[pl.ds(base, _CHUNK)], idx_v, sem1)
    cp_betas.wait()

    lane = lax.iota(jnp.int32, _L)
    shifts = [(jnp.maximum(lane - s, 0), lane >= s) for s in (1, 2, 4, 8)]
    last = jnp.full((_L,), _L - 1, jnp.int32)

    def prod_body(j, carry):
        v = 1.0 - betas_v[pl.ds(j * _L, _L)]
        for perm, mask in shifts:
            v = v * jnp.where(mask, _lane_perm(v, perm), 1.0)
        v = v * carry
        ab_v[pl.ds(j * _L, _L)] = v
        return _lane_perm(v, last)

    lax.fori_loop(0, _TPAD // _L, prod_body, jnp.ones((_L,), jnp.float32))
    cp_idx.wait()

    def gather_body(i, _):
        iv = idx_v[pl.ds(i * _L, _L)]
        b = plsc.load_gather(betas_v, [iv])
        p = plsc.load_gather(ab_v, [iv])
        ob_v[0, pl.ds(i * _L, _L)] = b
        oa_v[0, pl.ds(i * _L, _L)] = 1.0 - b
        op_v[0, pl.ds(i * _L, _L)] = p
        return 0

    lax.fori_loop(0, _CHUNK // _L, gather_body, 0)

    cp0 = pltpu.async_copy(ob_v, out_hbm.at[pl.ds(0, 1), pl.ds(base, _CHUNK)], sem0)
    cp1 = pltpu.async_copy(oa_v, out_hbm.at[pl.ds(1, 1), pl.ds(base, _CHUNK)], sem1)
    cp2 = pltpu.async_copy(op_v, out_hbm.at[pl.ds(2, 1), pl.ds(base, _CHUNK)], sem2)
    cp0.wait()
    cp1.wait()
    cp2.wait()


_sc_lookup = functools.partial(
    pl.kernel,
    mesh=plsc.VectorSubcoreMesh(core_axis_name="c", subcore_axis_name="s"),
    compiler_params=pltpu.CompilerParams(needs_layout_passes=False),
    out_type=jax.ShapeDtypeStruct((3, _B), jnp.float32),
    scratch_types=[
        pltpu.VMEM((_TPAD,), jnp.float32),
        pltpu.VMEM((_TPAD,), jnp.float32),
        pltpu.VMEM((_CHUNK,), jnp.int32),
        pltpu.VMEM((1, _CHUNK), jnp.float32),
        pltpu.VMEM((1, _CHUNK), jnp.float32),
        pltpu.VMEM((1, _CHUNK), jnp.float32),
        pltpu.SemaphoreType.DMA,
        pltpu.SemaphoreType.DMA,
        pltpu.SemaphoreType.DMA,
    ],
)(_sc_body)


def kernel(num_steps, betas):
    return _sc_lookup(num_steps.astype(jnp.int32), betas)
